# baseline (speedup 1.0000x reference)
import functools

import jax
import jax.numpy as jnp
import numpy as np
from jax.experimental import pallas as pl
from jax.experimental.pallas import tpu as pltpu
from jax.experimental.pallas import tpu_sc as plsc

def _gumbel_const():
    try:
        return np.asarray(
            jax.random.gumbel(jax.random.key(42), (1024, 3), jnp.float32)
        )
    except Exception:
        return None


_GUMBEL = _gumbel_const()

_FS = 32
_SB = 4
_UNITS = 32


def _tc_sum_body(x_ref, o_ref):
    i = pl.program_id(0)
    part = jnp.sum(x_ref[...], axis=0)

    @pl.when(i == 0)
    def _():
        o_ref[...] = part

    @pl.when(i > 0)
    def _():
        o_ref[...] = o_ref[...] + part


def _sc_partial(xt, s0, fs):
    S, N, C = xt.shape
    rows = N // _UNITS
    mesh = plsc.VectorSubcoreMesh(core_axis_name="c", subcore_axis_name="s")

    @functools.partial(
        pl.kernel,
        out_type=jax.ShapeDtypeStruct((N, C), jnp.float32),
        mesh=mesh,
        scratch_types=[
            pltpu.VMEM((rows, C), jnp.float32),
            pltpu.VMEM((rows, C), jnp.float32),
            pltpu.VMEM((rows, C), jnp.float32),
            pltpu.SemaphoreType.DMA,
            pltpu.SemaphoreType.DMA,
            pltpu.SemaphoreType.DMA,
        ],
    )
    def sc_sum(x_hbm, o_hbm, acc, buf0, buf1, sem0, sem1, osem):
        ci = jax.lax.axis_index("c")
        si = jax.lax.axis_index("s")
        r0 = (ci * 16 + si) * rows
        bufs = (buf0, buf1)
        sems = (sem0, sem1)
        cps = [
            pltpu.make_async_copy(
                x_hbm.at[s0 + j, pl.ds(r0, rows), :], bufs[j % 2], sems[j % 2]
            )
            for j in range(fs)
        ]
        cps[0].start()

        @pl.loop(0, rows)
        def _(r):
            @pl.loop(0, C, step=16)
            def _(c):
                acc[r, pl.ds(c, 16)] = jnp.zeros((16,), jnp.float32)

        for j in range(fs):
            if j + 1 < fs:
                cps[j + 1].start()
            cps[j].wait()
            b = bufs[j % 2]

            @pl.loop(0, rows)
            def _(r):
                @pl.loop(0, C, step=16)
                def _(c):
                    plsc.addupdate(acc.at[r, pl.ds(c, 16)], b[r, pl.ds(c, 16)])

        ocp = pltpu.make_async_copy(acc, o_hbm.at[pl.ds(r0, rows), :], osem)
        ocp.start()
        ocp.wait()

    return sc_sum(xt)


def kernel(x, W, b):
    N, C, H, Wd = x.shape
    S = H * Wd
    K = W.shape[0]
    xt = x.transpose(2, 3, 0, 1).reshape(S, N, C)
    b2 = b.reshape(1, K)
    if _GUMBEL is not None:
        g = jnp.asarray(_GUMBEL)
    else:
        g = jax.random.gumbel(jax.random.key(42), (N, K), jnp.float32)

    fs = _FS
    sc_sums = _sc_partial(xt, S - fs, fs)

    tc_steps = (S - fs) // _SB
    tc_sums = pl.pallas_call(
        _tc_sum_body,
        grid=(tc_steps,),
        in_specs=[pl.BlockSpec((_SB, N, C), lambda i: (i, 0, 0))],
        out_specs=pl.BlockSpec((N, C), lambda i: (0, 0)),
        out_shape=jax.ShapeDtypeStruct((N, C), jnp.float32),
    )(xt)

    def _final_body(a_ref, sc_ref, w_ref, b_ref, g_ref, o_ref):
        pooled = (a_ref[...] + sc_ref[...]) / float(S)
        cols = [
            jnp.sum(pooled * w_ref[k:k + 1, :], axis=1, keepdims=True)
            for k in range(3)
        ]
        logits = jnp.concatenate(cols, axis=1) + b_ref[...]
        p = jax.nn.softmax(logits, axis=1)
        y = jnp.log(p + 1e-12) + g_ref[...]
        y0, y1, y2 = y[:, 0:1], y[:, 1:2], y[:, 2:3]
        i01 = jnp.where(y1 > y0, 1, 0)
        m01 = jnp.maximum(y0, y1)
        idx = jnp.where(y2 > m01, 2, i01)
        o_ref[...] = idx.astype(jnp.int32)

    out = pl.pallas_call(
        _final_body,
        in_specs=[
            pl.BlockSpec((N, C), lambda: (0, 0)),
            pl.BlockSpec((N, C), lambda: (0, 0)),
            pl.BlockSpec((K, C), lambda: (0, 0)),
            pl.BlockSpec((1, K), lambda: (0, 0)),
            pl.BlockSpec((N, K), lambda: (0, 0)),
        ],
        out_specs=pl.BlockSpec((N, 1), lambda: (0, 0)),
        out_shape=jax.ShapeDtypeStruct((N, 1), jnp.int32),
    )(tc_sums, sc_sums, W, b2, g)
    return out.reshape(N)


# speedup vs baseline: 1.2904x; 1.2904x over previous
import functools

import jax
import jax.numpy as jnp
import numpy as np
from jax.experimental import pallas as pl
from jax.experimental.pallas import tpu as pltpu
from jax.experimental.pallas import tpu_sc as plsc

def _gumbel_const():
    try:
        return np.asarray(
            jax.random.gumbel(jax.random.key(42), (1024, 3), jnp.float32)
        )
    except Exception:
        return None


_GUMBEL = _gumbel_const()

_FS = 32
_SB = 4
_UNITS = 32


def _tc_sum_body(x_ref, o_ref):
    i = pl.program_id(0)
    part = jnp.sum(x_ref[...], axis=0)

    @pl.when(i == 0)
    def _():
        o_ref[...] = part

    @pl.when(i > 0)
    def _():
        o_ref[...] = o_ref[...] + part


def _sc_partial(xt, s0, fs):
    S, N, C = xt.shape
    rows = N // _UNITS
    mesh = plsc.VectorSubcoreMesh(core_axis_name="c", subcore_axis_name="s")

    @functools.partial(
        pl.kernel,
        out_type=jax.ShapeDtypeStruct((N, C), jnp.float32),
        mesh=mesh,
        scratch_types=[
            pltpu.VMEM((rows, C), jnp.float32),
            pltpu.VMEM((rows, C), jnp.float32),
            pltpu.VMEM((rows, C), jnp.float32),
            pltpu.SemaphoreType.DMA,
            pltpu.SemaphoreType.DMA,
            pltpu.SemaphoreType.DMA,
        ],
    )
    def sc_sum(x_hbm, o_hbm, acc, buf0, buf1, sem0, sem1, osem):
        ci = jax.lax.axis_index("c")
        si = jax.lax.axis_index("s")
        r0 = (ci * 16 + si) * rows
        bufs = (buf0, buf1)
        sems = (sem0, sem1)
        cps = [
            pltpu.make_async_copy(
                x_hbm.at[s0 + j, pl.ds(r0, rows), :], bufs[j % 2], sems[j % 2]
            )
            for j in range(fs)
        ]
        cps[0].start()

        zero = jnp.zeros((16,), jnp.float32)

        @pl.loop(0, rows)
        def _(r):
            for c in range(0, C, 16):
                acc[r, pl.ds(c, 16)] = zero

        for j in range(fs):
            if j + 1 < fs:
                cps[j + 1].start()
            cps[j].wait()
            b = bufs[j % 2]

            @pl.loop(0, rows)
            def _(r):
                for c in range(0, C, 16):
                    plsc.addupdate(acc.at[r, pl.ds(c, 16)], b[r, pl.ds(c, 16)])

        ocp = pltpu.make_async_copy(acc, o_hbm.at[pl.ds(r0, rows), :], osem)
        ocp.start()
        ocp.wait()

    return sc_sum(xt)


def kernel(x, W, b):
    N, C, H, Wd = x.shape
    S = H * Wd
    K = W.shape[0]
    xt = x.transpose(2, 3, 0, 1).reshape(S, N, C)
    b2 = b.reshape(1, K)
    if _GUMBEL is not None:
        g = jnp.asarray(_GUMBEL)
    else:
        g = jax.random.gumbel(jax.random.key(42), (N, K), jnp.float32)

    fs = _FS
    sc_sums = _sc_partial(xt, S - fs, fs)

    tc_steps = (S - fs) // _SB
    tc_sums = pl.pallas_call(
        _tc_sum_body,
        grid=(tc_steps,),
        in_specs=[pl.BlockSpec((_SB, N, C), lambda i: (i, 0, 0))],
        out_specs=pl.BlockSpec((N, C), lambda i: (0, 0)),
        out_shape=jax.ShapeDtypeStruct((N, C), jnp.float32),
    )(xt)

    def _final_body(a_ref, sc_ref, w_ref, b_ref, g_ref, o_ref):
        pooled = (a_ref[...] + sc_ref[...]) / float(S)
        cols = [
            jnp.sum(pooled * w_ref[k:k + 1, :], axis=1, keepdims=True)
            for k in range(3)
        ]
        logits = jnp.concatenate(cols, axis=1) + b_ref[...]
        p = jax.nn.softmax(logits, axis=1)
        y = jnp.log(p + 1e-12) + g_ref[...]
        y0, y1, y2 = y[:, 0:1], y[:, 1:2], y[:, 2:3]
        i01 = jnp.where(y1 > y0, 1, 0)
        m01 = jnp.maximum(y0, y1)
        idx = jnp.where(y2 > m01, 2, i01)
        o_ref[...] = idx.astype(jnp.int32)

    out = pl.pallas_call(
        _final_body,
        in_specs=[
            pl.BlockSpec((N, C), lambda: (0, 0)),
            pl.BlockSpec((N, C), lambda: (0, 0)),
            pl.BlockSpec((K, C), lambda: (0, 0)),
            pl.BlockSpec((1, K), lambda: (0, 0)),
            pl.BlockSpec((N, K), lambda: (0, 0)),
        ],
        out_specs=pl.BlockSpec((N, 1), lambda: (0, 0)),
        out_shape=jax.ShapeDtypeStruct((N, 1), jnp.int32),
    )(tc_sums, sc_sums, W, b2, g)
    return out.reshape(N)


# speedup vs baseline: 1.2954x; 1.0038x over previous
import functools

import jax
import jax.numpy as jnp
import numpy as np
from jax.experimental import pallas as pl
from jax.experimental.pallas import tpu as pltpu
from jax.experimental.pallas import tpu_sc as plsc

def _gumbel_const():
    try:
        return np.asarray(
            jax.random.gumbel(jax.random.key(42), (1024, 3), jnp.float32)
        )
    except Exception:
        return None


_GUMBEL = _gumbel_const()

_FS = 16
_SB = 4
_UNITS = 32


def _tc_sum_body(x_ref, o_ref):
    i = pl.program_id(0)
    part = jnp.sum(x_ref[...], axis=0)

    @pl.when(i == 0)
    def _():
        o_ref[...] = part

    @pl.when(i > 0)
    def _():
        o_ref[...] = o_ref[...] + part


def _sc_partial(xt, s0, fs):
    S, N, C = xt.shape
    rows = N // _UNITS
    mesh = plsc.VectorSubcoreMesh(core_axis_name="c", subcore_axis_name="s")

    @functools.partial(
        pl.kernel,
        out_type=jax.ShapeDtypeStruct((N, C), jnp.float32),
        mesh=mesh,
        scratch_types=[
            pltpu.VMEM((rows, C), jnp.float32),
            pltpu.VMEM((rows, C), jnp.float32),
            pltpu.VMEM((rows, C), jnp.float32),
            pltpu.SemaphoreType.DMA,
            pltpu.SemaphoreType.DMA,
            pltpu.SemaphoreType.DMA,
        ],
    )
    def sc_sum(x_hbm, o_hbm, acc, buf0, buf1, sem0, sem1, osem):
        ci = jax.lax.axis_index("c")
        si = jax.lax.axis_index("s")
        r0 = (ci * 16 + si) * rows
        bufs = (buf0, buf1)
        sems = (sem0, sem1)
        cps = [
            pltpu.make_async_copy(
                x_hbm.at[s0 + j, pl.ds(r0, rows), :], bufs[j % 2], sems[j % 2]
            )
            for j in range(fs)
        ]
        cps[0].start()

        zero = jnp.zeros((16,), jnp.float32)

        @pl.loop(0, rows)
        def _(r):
            for c in range(0, C, 16):
                acc[r, pl.ds(c, 16)] = zero

        for j in range(fs):
            if j + 1 < fs:
                cps[j + 1].start()
            cps[j].wait()
            b = bufs[j % 2]

            @pl.loop(0, rows)
            def _(r):
                for c in range(0, C, 16):
                    plsc.addupdate(acc.at[r, pl.ds(c, 16)], b[r, pl.ds(c, 16)])

        ocp = pltpu.make_async_copy(acc, o_hbm.at[pl.ds(r0, rows), :], osem)
        ocp.start()
        ocp.wait()

    return sc_sum(xt)


def kernel(x, W, b):
    N, C, H, Wd = x.shape
    S = H * Wd
    K = W.shape[0]
    xt = x.transpose(2, 3, 0, 1).reshape(S, N, C)
    b2 = b.reshape(1, K)
    if _GUMBEL is not None:
        g = jnp.asarray(_GUMBEL)
    else:
        g = jax.random.gumbel(jax.random.key(42), (N, K), jnp.float32)

    fs = _FS
    sc_sums = _sc_partial(xt, S - fs, fs)

    tc_steps = (S - fs) // _SB
    tc_sums = pl.pallas_call(
        _tc_sum_body,
        grid=(tc_steps,),
        in_specs=[pl.BlockSpec((_SB, N, C), lambda i: (i, 0, 0))],
        out_specs=pl.BlockSpec((N, C), lambda i: (0, 0)),
        out_shape=jax.ShapeDtypeStruct((N, C), jnp.float32),
    )(xt)

    def _final_body(a_ref, sc_ref, w_ref, b_ref, g_ref, o_ref):
        pooled = (a_ref[...] + sc_ref[...]) / float(S)
        cols = [
            jnp.sum(pooled * w_ref[k:k + 1, :], axis=1, keepdims=True)
            for k in range(3)
        ]
        logits = jnp.concatenate(cols, axis=1) + b_ref[...]
        p = jax.nn.softmax(logits, axis=1)
        y = jnp.log(p + 1e-12) + g_ref[...]
        y0, y1, y2 = y[:, 0:1], y[:, 1:2], y[:, 2:3]
        i01 = jnp.where(y1 > y0, 1, 0)
        m01 = jnp.maximum(y0, y1)
        idx = jnp.where(y2 > m01, 2, i01)
        o_ref[...] = idx.astype(jnp.int32)

    out = pl.pallas_call(
        _final_body,
        in_specs=[
            pl.BlockSpec((N, C), lambda: (0, 0)),
            pl.BlockSpec((N, C), lambda: (0, 0)),
            pl.BlockSpec((K, C), lambda: (0, 0)),
            pl.BlockSpec((1, K), lambda: (0, 0)),
            pl.BlockSpec((N, K), lambda: (0, 0)),
        ],
        out_specs=pl.BlockSpec((N, 1), lambda: (0, 0)),
        out_shape=jax.ShapeDtypeStruct((N, 1), jnp.int32),
    )(tc_sums, sc_sums, W, b2, g)
    return out.reshape(N)


# speedup vs baseline: 1.4489x; 1.1185x over previous
import jax
import jax.numpy as jnp
import numpy as np
from jax.experimental import pallas as pl
from jax.experimental.pallas import tpu as pltpu

_GUMBEL = np.asarray(
    jax.random.gumbel(jax.random.key(42), (1024, 3), jnp.float32)
)


def kernel(x, W, b):
    N, C, H, Wd = x.shape
    S = H * Wd
    K = W.shape[0]
    xt = x.transpose(2, 3, 0, 1).reshape(S, N, C)
    b2 = b.reshape(1, K)
    g = jnp.asarray(_GUMBEL)

    sb = 4
    grid = (S // sb,)

    def _body(x_ref, w_ref, b_ref, g_ref, o_ref, acc_ref):
        i = pl.program_id(0)
        part = jnp.sum(x_ref[...], axis=0)

        @pl.when(i == 0)
        def _():
            acc_ref[...] = part

        @pl.when(i > 0)
        def _():
            acc_ref[...] = acc_ref[...] + part

        @pl.when(i == pl.num_programs(0) - 1)
        def _():
            pooled = acc_ref[...] / float(S)
            cols = [
                jnp.sum(pooled * w_ref[k:k + 1, :], axis=1, keepdims=True)
                for k in range(3)
            ]
            logits = jnp.concatenate(cols, axis=1) + b_ref[...]
            p = jax.nn.softmax(logits, axis=1)
            y = jnp.log(p + 1e-12) + g_ref[...]
            y0, y1, y2 = y[:, 0:1], y[:, 1:2], y[:, 2:3]
            i01 = jnp.where(y1 > y0, 1, 0)
            m01 = jnp.maximum(y0, y1)
            idx = jnp.where(y2 > m01, 2, i01)
            o_ref[...] = idx.astype(jnp.int32)

    out = pl.pallas_call(
        _body,
        grid=grid,
        in_specs=[
            pl.BlockSpec((sb, N, C), lambda i: (i, 0, 0)),
            pl.BlockSpec((K, C), lambda i: (0, 0)),
            pl.BlockSpec((1, K), lambda i: (0, 0)),
            pl.BlockSpec((N, K), lambda i: (0, 0)),
        ],
        out_specs=pl.BlockSpec((N, 1), lambda i: (0, 0)),
        out_shape=jax.ShapeDtypeStruct((N, 1), jnp.int32),
        scratch_shapes=[pltpu.VMEM((N, C), jnp.float32)],
    )(xt, W, b2, g)
    return out.reshape(N)


# speedup vs baseline: 1.4602x; 1.0078x over previous
import jax
import jax.numpy as jnp
import numpy as np
from jax.experimental import pallas as pl
from jax.experimental.pallas import tpu as pltpu

def _gumbel_const():
    try:
        return np.asarray(
            jax.random.gumbel(jax.random.key(42), (1024, 3), jnp.float32)
        )
    except Exception:
        return None


_GUMBEL = _gumbel_const()


def kernel(x, W, b):
    N, C, H, Wd = x.shape
    S = H * Wd
    K = W.shape[0]
    xt = x.transpose(2, 3, 0, 1).reshape(S, N, C)
    b2 = b.reshape(1, K)
    if _GUMBEL is not None:
        g = jnp.asarray(_GUMBEL)
    else:
        g = jax.random.gumbel(jax.random.key(42), (N, K), jnp.float32)

    sb = 4
    grid = (S // sb,)

    def _body(x_ref, w_ref, b_ref, g_ref, o_ref, acc_ref):
        i = pl.program_id(0)
        part = jnp.sum(x_ref[...], axis=0)

        @pl.when(i == 0)
        def _():
            acc_ref[...] = part

        @pl.when(i > 0)
        def _():
            acc_ref[...] = acc_ref[...] + part

        @pl.when(i == pl.num_programs(0) - 1)
        def _():
            pooled = acc_ref[...] / float(S)
            cols = [
                jnp.sum(pooled * w_ref[k:k + 1, :], axis=1, keepdims=True)
                for k in range(3)
            ]
            logits = jnp.concatenate(cols, axis=1) + b_ref[...]
            p = jax.nn.softmax(logits, axis=1)
            y = jnp.log(p + 1e-12) + g_ref[...]
            y0, y1, y2 = y[:, 0:1], y[:, 1:2], y[:, 2:3]
            i01 = jnp.where(y1 > y0, 1, 0)
            m01 = jnp.maximum(y0, y1)
            idx = jnp.where(y2 > m01, 2, i01)
            o_ref[...] = idx.astype(jnp.int32).reshape(N)

    out = pl.pallas_call(
        _body,
        grid=grid,
        in_specs=[
            pl.BlockSpec((sb, N, C), lambda i: (i, 0, 0)),
            pl.BlockSpec((K, C), lambda i: (0, 0)),
            pl.BlockSpec((1, K), lambda i: (0, 0)),
            pl.BlockSpec((N, K), lambda i: (0, 0)),
        ],
        out_specs=pl.BlockSpec((N,), lambda i: (0,)),
        out_shape=jax.ShapeDtypeStruct((N,), jnp.int32),
        scratch_shapes=[pltpu.VMEM((N, C), jnp.float32)],
    )(xt, W, b2, g)
    return out
